# baseline (speedup 1.0000x reference)
import functools

import jax
import jax.numpy as jnp
from jax import lax
from jax.experimental import pallas as pl
from jax.experimental.pallas import tpu as pltpu
from jax.experimental.pallas import tpu_sc as plsc

N = 10000
D = 128
H = 64
C = 40

NC = 2
NS = 16
NW = NC * NS
CHUNK = 128
NACC = 10240
RPT = N // NS
DW = 16
NBUF = 12

_MESH = dict(core_axis_name="c", subcore_axis_name="s")


def _chunks(ept):
  out = []
  off = 0
  while off < ept:
    n = min(CHUNK, ept - off)
    out.append((off, n))
    off += n
  return out


def _fill(ref, nrows, width, value):
  v = jnp.full((16,), value, jnp.float32)

  def body(i, _):
    for k in range(width // 16):
      ref[i, pl.ds(k * 16, 16)] = v
    if width % 16:
      ref[i, pl.ds(width - 16, 16)] = v
    return 0

  lax.fori_loop(0, nrows, body, 0)


def _make_deg(e):
  ept = e // NW

  @functools.partial(
      pl.kernel,
      out_type=jax.ShapeDtypeStruct((NC, N, 8), jnp.float32),
      mesh=plsc.VectorSubcoreMesh(**_MESH),
      scratch_types=[
          pltpu.VMEM((ept,), jnp.int32),
          pltpu.VMEM((CHUNK, DW), jnp.float32),
          pltpu.VMEM_SHARED((NACC, DW), jnp.float32),
          pltpu.SemaphoreType.DMA,
      ],
      compiler_params=pltpu.CompilerParams(use_tc_tiling_on_sc=False),
  )
  def deg(ei_hbm, out_hbm, col_v, ones_v, acc, sem):
    c = lax.axis_index("c")
    s = lax.axis_index("s")
    wid = s * NC + c
    _fill(ones_v, CHUNK, DW, 1.0)
    for k in range(5):
      pltpu.sync_copy(ones_v.at[pl.ds(0, 125), :],
                      acc.at[pl.ds(s * RPT + k * 125, 125), :])
    plsc.subcore_barrier()
    pltpu.sync_copy(ei_hbm.at[1, pl.ds(wid * ept, ept)], col_v)

    K = 8
    descs = {}
    for i, (off, n) in enumerate(_chunks(ept)):
      descs[i] = pltpu.async_copy(
          ones_v.at[pl.ds(0, n), :],
          acc.at[col_v.at[pl.ds(off, n)]], sem, add=True)
      if i >= K:
        descs.pop(i - K).wait()
    for i in sorted(descs):
      descs.pop(i).wait()
    plsc.subcore_barrier()
    pltpu.sync_copy(acc.at[pl.ds(s * RPT, RPT), pl.ds(0, 8)],
                    out_hbm.at[c, pl.ds(s * RPT, RPT), :])

  return deg


def _make_agg(width, e, self_loop):
  ept = e // NW
  nbuf = NBUF if width <= 32 else 8

  @functools.partial(
      pl.kernel,
      out_type=jax.ShapeDtypeStruct((NC, N, width), jnp.float32),
      mesh=plsc.VectorSubcoreMesh(**_MESH),
      scratch_types=[
          pltpu.VMEM((ept,), jnp.int32),
          pltpu.VMEM((ept,), jnp.int32),
          pltpu.VMEM((nbuf, CHUNK, width), jnp.float32),
          pltpu.VMEM_SHARED((NACC, width), jnp.float32),
          pltpu.VMEM_SHARED((NACC, width), jnp.float32),
          [pltpu.SemaphoreType.DMA] * nbuf,
          [pltpu.SemaphoreType.DMA] * nbuf,
      ],
      compiler_params=pltpu.CompilerParams(use_tc_tiling_on_sc=False),
  )
  def agg(h_hbm, ei_hbm, out_hbm, row_v, col_v, rows_v, acc, table,
          gsems, ssems):
    c = lax.axis_index("c")
    s = lax.axis_index("s")
    wid = s * NC + c
    pltpu.sync_copy(h_hbm.at[pl.ds(s * RPT, RPT), :],
                    table.at[pl.ds(s * RPT, RPT), :])
    _fill(rows_v.at[0], CHUNK, width, 0.0)
    if self_loop:
      @pl.when(c == 0)
      def _():
        pltpu.sync_copy(h_hbm.at[pl.ds(s * RPT, RPT), :],
                        acc.at[pl.ds(s * RPT, RPT), :])

      @pl.when(c != 0)
      def _():
        for k in range(5):
          pltpu.sync_copy(rows_v.at[0, pl.ds(0, 125), :],
                          acc.at[pl.ds(s * RPT + k * 125, 125), :])
    else:
      for k in range(5):
        pltpu.sync_copy(rows_v.at[0, pl.ds(0, 125), :],
                        acc.at[pl.ds(s * RPT + k * 125, 125), :])
    pltpu.sync_copy(rows_v.at[0, pl.ds(0, 15), :],
                    acc.at[pl.ds(N + s * 15, 15), :])
    plsc.subcore_barrier()
    pltpu.sync_copy(ei_hbm.at[0, pl.ds(wid * ept, ept)], row_v)
    pltpu.sync_copy(ei_hbm.at[1, pl.ds(wid * ept, ept)], col_v)

    chunks = _chunks(ept)
    nch = len(chunks)
    gd = {}
    sd = {}

    def fire_gather(m):
      b = m % nbuf
      off, n = chunks[m]
      gd[m] = pltpu.async_copy(
          table.at[row_v.at[pl.ds(off, n)]],
          rows_v.at[b, pl.ds(0, n), :], gsems[b])

    def fire_scatter(m):
      b = m % nbuf
      off, n = chunks[m]
      sd[m] = pltpu.async_copy(
          rows_v.at[b, pl.ds(0, n), :],
          acc.at[col_v.at[pl.ds(off, n)]], ssems[b], add=True)

    for m in range(min(nbuf, nch)):
      fire_gather(m)
    for j in range(nch):
      gd.pop(j).wait()
      fire_scatter(j)
      jp = j - nbuf // 2
      if jp >= 0 and jp + nbuf < nch:
        sd.pop(jp).wait()
        fire_gather(jp + nbuf)
    for m in sorted(sd):
      sd.pop(m).wait()
    plsc.subcore_barrier()
    pltpu.sync_copy(acc.at[pl.ds(s * RPT, RPT), :],
                    out_hbm.at[c, pl.ds(s * RPT, RPT), :])

  return agg


def _tc1_body(deg_ref, x_ref, w1_ref, ha_ref, hb_ref, dinv_ref):
  deg = deg_ref[0, :, 0:1] + deg_ref[1, :, 0:1] - 1.0
  dinv = lax.rsqrt(deg)
  dinv_ref[...] = dinv
  h = jnp.dot(x_ref[...], w1_ref[...], preferred_element_type=jnp.float32)
  h = h * dinv
  ha_ref[...] = h[:, :H // 2]
  hb_ref[...] = h[:, H // 2:]


def _tc2_body(acca_ref, accb_ref, dinv_ref, b1_ref, w2_ref, h2_ref):
  dinv = dinv_ref[...]
  agg = jnp.concatenate(
      [acca_ref[0] + acca_ref[1], accb_ref[0] + accb_ref[1]], axis=1)
  z = jnp.maximum(agg * dinv + b1_ref[...], 0.0)
  h2 = jnp.dot(z, w2_ref[...], preferred_element_type=jnp.float32)
  h2_ref[...] = h2 * dinv


def _tc3_body(acc_ref, dinv_ref, b2_ref, out_ref):
  agg = acc_ref[0] + acc_ref[1]
  out_ref[...] = agg * dinv_ref[...] + b2_ref[...]


def kernel(x, edge_index, W1, b1, W2, b2):
  e = edge_index.shape[1]
  ei = edge_index.astype(jnp.int32)

  deg2 = _make_deg(e)(ei)

  h1a, h1b, dinvp = pl.pallas_call(
      _tc1_body,
      out_shape=[
          jax.ShapeDtypeStruct((N, H // 2), jnp.float32),
          jax.ShapeDtypeStruct((N, H // 2), jnp.float32),
          jax.ShapeDtypeStruct((N, 1), jnp.float32),
      ],
  )(deg2, x, W1)

  agg32 = _make_agg(H // 2, e, True)
  acc1a = agg32(h1a, ei)
  acc1b = agg32(h1b, ei)

  h2p = pl.pallas_call(
      _tc2_body,
      out_shape=jax.ShapeDtypeStruct((N, C), jnp.float32),
  )(acc1a, acc1b, dinvp, b1.reshape(1, H), W2)

  acc2 = _make_agg(C, e, True)(h2p, ei)

  return pl.pallas_call(
      _tc3_body,
      out_shape=jax.ShapeDtypeStruct((N, C), jnp.float32),
  )(acc2, dinvp, b2.reshape(1, C))


# speedup vs baseline: 1.0243x; 1.0243x over previous
import functools

import jax
import jax.numpy as jnp
from jax import lax
from jax.experimental import pallas as pl
from jax.experimental.pallas import tpu as pltpu
from jax.experimental.pallas import tpu_sc as plsc

N = 10000
D = 128
H = 64
C = 40

NC = 2
NS = 16
NW = NC * NS
CHUNK = 128
NACC = 10240
RPT = N // NS
DW = 16
NBUF = 12

_MESH = dict(core_axis_name="c", subcore_axis_name="s")


def _chunks(ept):
  out = []
  off = 0
  while off < ept:
    n = min(CHUNK, ept - off)
    out.append((off, n))
    off += n
  return out


def _fill(ref, nrows, width, value):
  v = jnp.full((16,), value, jnp.float32)

  def body(i, _):
    for k in range(width // 16):
      ref[i, pl.ds(k * 16, 16)] = v
    if width % 16:
      ref[i, pl.ds(width - 16, 16)] = v
    return 0

  lax.fori_loop(0, nrows, body, 0)


def _make_deg(e):
  ept = e // NW

  @functools.partial(
      pl.kernel,
      out_type=jax.ShapeDtypeStruct((NC, N, DW), jnp.float32),
      mesh=plsc.VectorSubcoreMesh(**_MESH),
      scratch_types=[
          pltpu.VMEM((ept,), jnp.int32),
          pltpu.VMEM((CHUNK, DW), jnp.float32),
          pltpu.VMEM_SHARED((NACC, DW), jnp.float32),
          pltpu.SemaphoreType.DMA,
      ],
      compiler_params=pltpu.CompilerParams(use_tc_tiling_on_sc=False),
  )
  def deg(ei_hbm, out_hbm, col_v, ones_v, acc, sem):
    c = lax.axis_index("c")
    s = lax.axis_index("s")
    wid = s * NC + c
    _fill(ones_v, CHUNK, DW, 1.0)
    for k in range(5):
      pltpu.sync_copy(ones_v.at[pl.ds(0, 125), :],
                      acc.at[pl.ds(s * RPT + k * 125, 125), :])
    plsc.subcore_barrier()
    pltpu.sync_copy(ei_hbm.at[1, pl.ds(wid * ept, ept)], col_v)

    K = 8
    descs = {}
    for i, (off, n) in enumerate(_chunks(ept)):
      descs[i] = pltpu.async_copy(
          ones_v.at[pl.ds(0, n), :],
          acc.at[col_v.at[pl.ds(off, n)]], sem, add=True)
      if i >= K:
        descs.pop(i - K).wait()
    for i in sorted(descs):
      descs.pop(i).wait()
    plsc.subcore_barrier()
    pltpu.sync_copy(acc.at[pl.ds(s * RPT, RPT), :],
                    out_hbm.at[c, pl.ds(s * RPT, RPT), :])

  return deg


def _make_agg(width, e, self_loop):
  ept = e // NW
  nbuf = NBUF if width <= 32 else 8

  @functools.partial(
      pl.kernel,
      out_type=jax.ShapeDtypeStruct((NC, N, width), jnp.float32),
      mesh=plsc.VectorSubcoreMesh(**_MESH),
      scratch_types=[
          pltpu.VMEM((ept,), jnp.int32),
          pltpu.VMEM((ept,), jnp.int32),
          pltpu.VMEM((nbuf, CHUNK, width), jnp.float32),
          pltpu.VMEM_SHARED((NACC, width), jnp.float32),
          pltpu.VMEM_SHARED((N, width), jnp.float32),
          [pltpu.SemaphoreType.DMA] * nbuf,
          [pltpu.SemaphoreType.DMA] * nbuf,
      ],
      compiler_params=pltpu.CompilerParams(use_tc_tiling_on_sc=False),
  )
  def agg(h_hbm, ei_hbm, out_hbm, row_v, col_v, rows_v, acc, table,
          gsems, ssems):
    c = lax.axis_index("c")
    s = lax.axis_index("s")
    wid = s * NC + c
    pltpu.sync_copy(h_hbm.at[pl.ds(s * RPT, RPT), :],
                    table.at[pl.ds(s * RPT, RPT), :])
    _fill(rows_v.at[0], CHUNK, width, 0.0)
    if self_loop:
      @pl.when(c == 0)
      def _():
        pltpu.sync_copy(h_hbm.at[pl.ds(s * RPT, RPT), :],
                        acc.at[pl.ds(s * RPT, RPT), :])

      @pl.when(c != 0)
      def _():
        for k in range(5):
          pltpu.sync_copy(rows_v.at[0, pl.ds(0, 125), :],
                          acc.at[pl.ds(s * RPT + k * 125, 125), :])
    else:
      for k in range(5):
        pltpu.sync_copy(rows_v.at[0, pl.ds(0, 125), :],
                        acc.at[pl.ds(s * RPT + k * 125, 125), :])
    pltpu.sync_copy(rows_v.at[0, pl.ds(0, 15), :],
                    acc.at[pl.ds(N + s * 15, 15), :])
    plsc.subcore_barrier()
    pltpu.sync_copy(ei_hbm.at[0, pl.ds(wid * ept, ept)], row_v)
    pltpu.sync_copy(ei_hbm.at[1, pl.ds(wid * ept, ept)], col_v)

    chunks = _chunks(ept)
    nch = len(chunks)
    gd = {}
    sd = {}

    def fire_gather(m):
      b = m % nbuf
      off, n = chunks[m]
      gd[m] = pltpu.async_copy(
          table.at[row_v.at[pl.ds(off, n)]],
          rows_v.at[b, pl.ds(0, n), :], gsems[b])

    def fire_scatter(m):
      b = m % nbuf
      off, n = chunks[m]
      sd[m] = pltpu.async_copy(
          rows_v.at[b, pl.ds(0, n), :],
          acc.at[col_v.at[pl.ds(off, n)]], ssems[b], add=True)

    for m in range(min(nbuf, nch)):
      fire_gather(m)
    for j in range(nch):
      gd.pop(j).wait()
      fire_scatter(j)
      jp = j - nbuf // 2
      if jp >= 0 and jp + nbuf < nch:
        sd.pop(jp).wait()
        fire_gather(jp + nbuf)
    for m in sorted(sd):
      sd.pop(m).wait()
    plsc.subcore_barrier()
    pltpu.sync_copy(acc.at[pl.ds(s * RPT, RPT), :],
                    out_hbm.at[c, pl.ds(s * RPT, RPT), :])

  return agg


def _tc1_body(deg_ref, x_ref, w1_ref, ha_ref, hb_ref, dinv_ref):
  deg = deg_ref[0, :, 0:1] + deg_ref[1, :, 0:1] - 1.0
  dinv = lax.rsqrt(deg)
  dinv_ref[...] = dinv
  h = jnp.dot(x_ref[...], w1_ref[...], preferred_element_type=jnp.float32)
  h = h * dinv
  ha_ref[...] = h[:, :H // 2]
  hb_ref[...] = h[:, H // 2:]


def _tc2_body(acca_ref, accb_ref, dinv_ref, b1_ref, w2_ref, h2_ref):
  dinv = dinv_ref[...]
  agg = jnp.concatenate(
      [acca_ref[0] + acca_ref[1], accb_ref[0] + accb_ref[1]], axis=1)
  z = jnp.maximum(agg * dinv + b1_ref[...], 0.0)
  h2 = jnp.dot(z, w2_ref[...], preferred_element_type=jnp.float32)
  h2_ref[...] = h2 * dinv


def _tc3_body(acc_ref, dinv_ref, b2_ref, out_ref):
  agg = acc_ref[0] + acc_ref[1]
  out_ref[...] = agg * dinv_ref[...] + b2_ref[...]


def kernel(x, edge_index, W1, b1, W2, b2):
  e = edge_index.shape[1]
  ei = edge_index.astype(jnp.int32)

  deg2 = _make_deg(e)(ei)

  h1a, h1b, dinvp = pl.pallas_call(
      _tc1_body,
      out_shape=[
          jax.ShapeDtypeStruct((N, H // 2), jnp.float32),
          jax.ShapeDtypeStruct((N, H // 2), jnp.float32),
          jax.ShapeDtypeStruct((N, 1), jnp.float32),
      ],
  )(deg2, x, W1)

  agg32 = _make_agg(H // 2, e, True)
  acc1a = agg32(h1a, ei)
  acc1b = agg32(h1b, ei)

  h2p = pl.pallas_call(
      _tc2_body,
      out_shape=jax.ShapeDtypeStruct((N, C), jnp.float32),
  )(acc1a, acc1b, dinvp, b1.reshape(1, H), W2)

  acc2 = _make_agg(C, e, True)(h2p, ei)

  return pl.pallas_call(
      _tc3_body,
      out_shape=jax.ShapeDtypeStruct((N, C), jnp.float32),
  )(acc2, dinvp, b2.reshape(1, C))


# speedup vs baseline: 1.0261x; 1.0017x over previous
import functools

import jax
import jax.numpy as jnp
from jax import lax
from jax.experimental import pallas as pl
from jax.experimental.pallas import tpu as pltpu
from jax.experimental.pallas import tpu_sc as plsc

N = 10000
D = 128
H = 64
C = 40

NC = 2
NS = 16
NW = NC * NS
CHUNK = 128
NACC = 10240
RPT = N // NS
DW = 16
NBUF = 12

_MESH = dict(core_axis_name="c", subcore_axis_name="s")


def _chunks(ept):
  out = []
  off = 0
  while off < ept:
    n = min(CHUNK, ept - off)
    out.append((off, n))
    off += n
  return out


def _fill(ref, nrows, width, value):
  v = jnp.full((16,), value, jnp.float32)

  def body(i, _):
    for k in range(width // 16):
      ref[i, pl.ds(k * 16, 16)] = v
    if width % 16:
      ref[i, pl.ds(width - 16, 16)] = v
    return 0

  lax.fori_loop(0, nrows, body, 0)


def _make_deg(e):
  ept = e // NW

  @functools.partial(
      pl.kernel,
      out_type=jax.ShapeDtypeStruct((NC, N, DW), jnp.float32),
      mesh=plsc.VectorSubcoreMesh(**_MESH),
      scratch_types=[
          pltpu.VMEM((ept,), jnp.int32),
          pltpu.VMEM((CHUNK, DW), jnp.float32),
          pltpu.VMEM_SHARED((NACC, DW), jnp.float32),
          pltpu.SemaphoreType.DMA,
      ],
      compiler_params=pltpu.CompilerParams(use_tc_tiling_on_sc=False),
  )
  def deg(ei_hbm, out_hbm, col_v, ones_v, acc, sem):
    c = lax.axis_index("c")
    s = lax.axis_index("s")
    wid = s * NC + c
    _fill(ones_v, CHUNK, DW, 1.0)
    for k in range(5):
      pltpu.sync_copy(ones_v.at[pl.ds(0, 125), :],
                      acc.at[pl.ds(s * RPT + k * 125, 125), :])
    plsc.subcore_barrier()
    pltpu.sync_copy(ei_hbm.at[1, pl.ds(wid * ept, ept)], col_v)

    K = 8
    descs = {}
    for i, (off, n) in enumerate(_chunks(ept)):
      descs[i] = pltpu.async_copy(
          ones_v.at[pl.ds(0, n), :],
          acc.at[col_v.at[pl.ds(off, n)]], sem, add=True)
      if i >= K:
        descs.pop(i - K).wait()
    for i in sorted(descs):
      descs.pop(i).wait()
    plsc.subcore_barrier()
    pltpu.sync_copy(acc.at[pl.ds(s * RPT, RPT), :],
                    out_hbm.at[c, pl.ds(s * RPT, RPT), :])

  return deg


def _make_agg(width, e, self_loop):
  ept = e // NW
  nbuf = NBUF if width <= 32 else 8

  @functools.partial(
      pl.kernel,
      out_type=jax.ShapeDtypeStruct((NC, N, width), jnp.float32),
      mesh=plsc.VectorSubcoreMesh(**_MESH),
      scratch_types=[
          pltpu.VMEM((ept,), jnp.int32),
          pltpu.VMEM((ept,), jnp.int32),
          pltpu.VMEM((nbuf, CHUNK, width), jnp.float32),
          pltpu.VMEM_SHARED((NACC, width), jnp.float32),
          pltpu.VMEM_SHARED((N, width), jnp.float32),
          [pltpu.SemaphoreType.DMA] * nbuf,
          [pltpu.SemaphoreType.DMA] * nbuf,
      ],
      compiler_params=pltpu.CompilerParams(use_tc_tiling_on_sc=False),
  )
  def agg(h_hbm, ei_hbm, out_hbm, row_v, col_v, rows_v, acc, table,
          gsems, ssems):
    c = lax.axis_index("c")
    s = lax.axis_index("s")
    wid = s * NC + c
    pltpu.sync_copy(h_hbm.at[pl.ds(s * RPT, RPT), :],
                    table.at[pl.ds(s * RPT, RPT), :])
    _fill(rows_v.at[0], CHUNK, width, 0.0)
    if self_loop:
      @pl.when(c == 0)
      def _():
        pltpu.sync_copy(h_hbm.at[pl.ds(s * RPT, RPT), :],
                        acc.at[pl.ds(s * RPT, RPT), :])

      @pl.when(c != 0)
      def _():
        for k in range(5):
          pltpu.sync_copy(rows_v.at[0, pl.ds(0, 125), :],
                          acc.at[pl.ds(s * RPT + k * 125, 125), :])
    else:
      for k in range(5):
        pltpu.sync_copy(rows_v.at[0, pl.ds(0, 125), :],
                        acc.at[pl.ds(s * RPT + k * 125, 125), :])
    pltpu.sync_copy(rows_v.at[0, pl.ds(0, 15), :],
                    acc.at[pl.ds(N + s * 15, 15), :])
    plsc.subcore_barrier()
    pltpu.sync_copy(ei_hbm.at[0, pl.ds(wid * ept, ept)], row_v)
    pltpu.sync_copy(ei_hbm.at[1, pl.ds(wid * ept, ept)], col_v)

    chunks = _chunks(ept)
    nch = len(chunks)
    gd = {}
    sd = {}

    def fire_gather(m):
      b = m % nbuf
      off, n = chunks[m]
      gd[m] = pltpu.async_copy(
          table.at[row_v.at[pl.ds(off, n)]],
          rows_v.at[b, pl.ds(0, n), :], gsems[b])

    def fire_scatter(m):
      b = m % nbuf
      off, n = chunks[m]
      sd[m] = pltpu.async_copy(
          rows_v.at[b, pl.ds(0, n), :],
          acc.at[col_v.at[pl.ds(off, n)]], ssems[b], add=True)

    for m in range(min(nbuf, nch)):
      fire_gather(m)
    for j in range(nch):
      gd.pop(j).wait()
      fire_scatter(j)
      jp = j - nbuf // 2
      if jp >= 0 and jp + nbuf < nch:
        sd.pop(jp).wait()
        fire_gather(jp + nbuf)
    for m in sorted(sd):
      sd.pop(m).wait()
    plsc.subcore_barrier()
    pltpu.sync_copy(acc.at[pl.ds(s * RPT, RPT), :],
                    out_hbm.at[c, pl.ds(s * RPT, RPT), :])

  return agg


def _mm1_body(x_ref, w1_ref, h_ref):
  h_ref[...] = jnp.dot(x_ref[...], w1_ref[...],
                       preferred_element_type=jnp.float32)


def _tc1_body(deg_ref, h_ref, ha_ref, hb_ref, dinv_ref):
  deg = deg_ref[0, :, 0:1] + deg_ref[1, :, 0:1] - 1.0
  dinv = lax.rsqrt(deg)
  dinv_ref[...] = dinv
  h = h_ref[...] * dinv
  ha_ref[...] = h[:, :H // 2]
  hb_ref[...] = h[:, H // 2:]


def _tc2_body(acca_ref, accb_ref, dinv_ref, b1_ref, w2_ref, h2_ref):
  dinv = dinv_ref[...]
  agg = jnp.concatenate(
      [acca_ref[0] + acca_ref[1], accb_ref[0] + accb_ref[1]], axis=1)
  z = jnp.maximum(agg * dinv + b1_ref[...], 0.0)
  h2 = jnp.dot(z, w2_ref[...], preferred_element_type=jnp.float32)
  h2_ref[...] = h2 * dinv


def _tc3_body(acc_ref, dinv_ref, b2_ref, out_ref):
  agg = acc_ref[0] + acc_ref[1]
  out_ref[...] = agg * dinv_ref[...] + b2_ref[...]


def kernel(x, edge_index, W1, b1, W2, b2):
  e = edge_index.shape[1]
  ei = edge_index.astype(jnp.int32)

  h1 = pl.pallas_call(
      _mm1_body,
      out_shape=jax.ShapeDtypeStruct((N, H), jnp.float32),
  )(x, W1)

  deg2 = _make_deg(e)(ei)

  h1a, h1b, dinvp = pl.pallas_call(
      _tc1_body,
      out_shape=[
          jax.ShapeDtypeStruct((N, H // 2), jnp.float32),
          jax.ShapeDtypeStruct((N, H // 2), jnp.float32),
          jax.ShapeDtypeStruct((N, 1), jnp.float32),
      ],
  )(deg2, h1)

  agg32 = _make_agg(H // 2, e, True)
  acc1a = agg32(h1a, ei)
  acc1b = agg32(h1b, ei)

  h2p = pl.pallas_call(
      _tc2_body,
      out_shape=jax.ShapeDtypeStruct((N, C), jnp.float32),
  )(acc1a, acc1b, dinvp, b1.reshape(1, H), W2)

  acc2 = _make_agg(C, e, True)(h2p, ei)

  return pl.pallas_call(
      _tc3_body,
      out_shape=jax.ShapeDtypeStruct((N, C), jnp.float32),
  )(acc2, dinvp, b2.reshape(1, C))
